# baseline (speedup 1.0000x reference)
import jax
import jax.numpy as jnp
from jax import lax
from jax.experimental import pallas as pl
from jax.experimental.pallas import tpu as pltpu
from jax.experimental.pallas import tpu_sc as plsc

VOCAB = 1000000
D = 64
NI, NJ = 4096, 200
B = NI * NJ
NC, NS = 2, 16
NW = NC * NS

W1 = 1024
G1 = (VOCAB + 2 * W1 - 1) // (2 * W1)
TPACK_ROWS = G1 * W1


def _repack_body(a_ref, b_ref, o_ref):
    ta = jnp.transpose(a_ref[...])
    tb = jnp.transpose(b_ref[...])
    o_ref[...] = jnp.concatenate([ta, tb], axis=1)


_repack = pl.pallas_call(
    _repack_body,
    grid=(G1,),
    in_specs=[
        pl.BlockSpec((D, W1), lambda i: (0, 2 * i)),
        pl.BlockSpec((D, W1), lambda i: (0, jnp.minimum(2 * i + 1, 975))),
    ],
    out_specs=pl.BlockSpec((W1, 128), lambda i: (i, 0)),
    out_shape=jax.ShapeDtypeStruct((TPACK_ROWS, 128), jnp.float32),
)

CHUNK = 1024


def _gather_body(idx_hbm, tview_hbm, out_hbm, idx_v, rows_v, sem):
    nrows = idx_hbm.shape[0]
    per_w = nrows // NW
    n_chunks = per_w // CHUNK
    wid = lax.axis_index("s") * NC + lax.axis_index("c")
    base = wid * per_w

    def step(i, carry):
        off = base + i * CHUNK
        pltpu.sync_copy(idx_hbm.at[pl.ds(off, CHUNK)], idx_v)
        pltpu.async_copy(tview_hbm.at[idx_v], rows_v, sem).wait()
        pltpu.sync_copy(rows_v, out_hbm.at[pl.ds(off, CHUNK)])
        return carry

    lax.fori_loop(0, n_chunks, step, 0)


def _make_gather(nrows):
    return pl.kernel(
        _gather_body,
        out_type=jax.ShapeDtypeStruct((nrows, D), jnp.float32),
        mesh=plsc.VectorSubcoreMesh(core_axis_name="c", subcore_axis_name="s"),
        compiler_params=pltpu.CompilerParams(use_tc_tiling_on_sc=False),
        scratch_types=[
            pltpu.VMEM((CHUNK,), jnp.int32),
            pltpu.VMEM((CHUNK, D), jnp.float32),
            pltpu.SemaphoreType.DMA,
        ],
    )


NQ = NI // 2
NJ_SPLITS = (48, 48, 48, 56)


def _select_t_body(r_ref, o_ref):
    blk = r_ref[...]
    tblk = jnp.transpose(blk, (0, 2, 1))
    o_ref[...] = jnp.concatenate([tblk[:, :D, :], tblk[:, D:, :]], axis=2)


def _select_t_body_aliased(r_ref, _prev_ref, o_ref):
    _select_t_body(r_ref, o_ref)


_OUT3_TYPE = jax.ShapeDtypeStruct((NJ, D, NI), jnp.float32)


def _make_select(nj_split, j_off, aliased):
    in_specs = [pl.BlockSpec((1, NQ, 128), lambda a: (a, 0, 0))]
    kwargs = {}
    body = _select_t_body
    if aliased:
        in_specs.append(pl.BlockSpec(memory_space=pl.ANY))
        kwargs["input_output_aliases"] = {1: 0}
        body = _select_t_body_aliased
    return pl.pallas_call(
        body,
        grid=(nj_split,),
        in_specs=in_specs,
        out_specs=pl.BlockSpec((1, D, NI), lambda a: (a + j_off, 0, 0)),
        out_shape=_OUT3_TYPE,
        **kwargs,
    )


def kernel(x, table):
    tT = jnp.transpose(table)
    xt = jnp.transpose(x).astype(jnp.int32)
    tpack = _repack(tT, tT)
    tview = jnp.reshape(tpack, (2 * TPACK_ROWS, D))

    g = ((xt >> 11) << 11) + ((xt & 1023) << 1) + ((xt >> 10) & 1)
    ga, gb = g[:, :NQ], g[:, NQ:]
    even = (lax.broadcasted_iota(jnp.int32, (NJ, NI), 1) & 1) == 0
    idx3 = jnp.where(
        even, jnp.repeat(ga, 2, axis=1), jnp.repeat(gb, 2, axis=1)
    ).reshape(-1)

    out3 = None
    j_off = 0
    for nj_split in NJ_SPLITS:
        rows = nj_split * NI
        r0 = j_off * NI
        idx_s = lax.slice(idx3, (r0,), (r0 + rows,))
        out2 = _make_gather(rows)(idx_s, tview)
        r3 = jnp.reshape(out2, (nj_split, NQ, 128))
        sel = _make_select(nj_split, j_off, aliased=out3 is not None)
        out3 = sel(r3) if out3 is None else sel(r3, out3)
        j_off += nj_split
    return jnp.transpose(out3, (2, 0, 1))


# speedup vs baseline: 1.5482x; 1.5482x over previous
import jax
import jax.numpy as jnp
from jax import lax
from jax.experimental import pallas as pl
from jax.experimental.pallas import tpu as pltpu
from jax.experimental.pallas import tpu_sc as plsc

VOCAB = 1000000
D = 64
NI, NJ = 4096, 200
B = NI * NJ
NC, NS = 2, 16
NW = NC * NS

W1 = 1024
G1 = (VOCAB + 2 * W1 - 1) // (2 * W1)
TPACK_ROWS = G1 * W1


def _repack_body(a_ref, b_ref, o_ref):
    ta = jnp.transpose(a_ref[...])
    tb = jnp.transpose(b_ref[...])
    o_ref[...] = jnp.concatenate([ta, tb], axis=1)


_repack = pl.pallas_call(
    _repack_body,
    grid=(G1,),
    in_specs=[
        pl.BlockSpec((D, W1), lambda i: (0, 2 * i)),
        pl.BlockSpec((D, W1), lambda i: (0, jnp.minimum(2 * i + 1, 975))),
    ],
    out_specs=pl.BlockSpec((W1, 128), lambda i: (i, 0)),
    out_shape=jax.ShapeDtypeStruct((TPACK_ROWS, 128), jnp.float32),
)

CHUNK = 1024


def _gather_body(idx_hbm, tview_hbm, out_hbm, idx_v, rows_v, sem):
    n_chunks_total = idx_hbm.shape[0] // CHUNK
    per_w = n_chunks_total // NW
    wid = lax.axis_index("s") * NC + lax.axis_index("c")

    def step(i, carry):
        t = wid * per_w + i
        j = t >> 2
        s = (t >> 1) & 1
        c = t & 1
        pltpu.sync_copy(
            idx_hbm.at[pl.ds(j * NI + s * NQ + c * CHUNK, CHUNK)], idx_v
        )
        pltpu.async_copy(tview_hbm.at[idx_v], rows_v, sem).wait()
        pltpu.sync_copy(
            rows_v,
            out_hbm.at[pl.ds(j * NQ + c * CHUNK, CHUNK), pl.ds(s * D, D)],
        )
        return carry

    lax.fori_loop(0, per_w, step, 0)


def _make_gather(nrows):
    return pl.kernel(
        _gather_body,
        out_type=jax.ShapeDtypeStruct((nrows // 2, 2 * D), jnp.float32),
        mesh=plsc.VectorSubcoreMesh(core_axis_name="c", subcore_axis_name="s"),
        compiler_params=pltpu.CompilerParams(use_tc_tiling_on_sc=False),
        scratch_types=[
            pltpu.VMEM((CHUNK,), jnp.int32),
            pltpu.VMEM((CHUNK, D), jnp.float32),
            pltpu.SemaphoreType.DMA,
        ],
    )


NQ = NI // 2
NJ_SPLITS = (48, 48, 48, 56)


def _select_t_body(r_ref, o_ref):
    blk = r_ref[...]
    tblk = jnp.transpose(blk, (0, 2, 1))
    o_ref[...] = jnp.concatenate([tblk[:, :D, :], tblk[:, D:, :]], axis=2)


def _select_t_body_aliased(r_ref, _prev_ref, o_ref):
    _select_t_body(r_ref, o_ref)


_OUT3_TYPE = jax.ShapeDtypeStruct((NJ, D, NI), jnp.float32)


def _make_select(nj_split, j_off, aliased):
    in_specs = [pl.BlockSpec((1, NQ, 128), lambda a: (a, 0, 0))]
    kwargs = {}
    body = _select_t_body
    if aliased:
        in_specs.append(pl.BlockSpec(memory_space=pl.ANY))
        kwargs["input_output_aliases"] = {1: 0}
        body = _select_t_body_aliased
    return pl.pallas_call(
        body,
        grid=(nj_split,),
        in_specs=in_specs,
        out_specs=pl.BlockSpec((1, D, NI), lambda a: (a + j_off, 0, 0)),
        out_shape=_OUT3_TYPE,
        **kwargs,
    )


def kernel(x, table):
    tT = jnp.transpose(table)
    xt = jnp.transpose(x).astype(jnp.int32)
    tpack = _repack(tT, tT)
    tview = jnp.reshape(tpack, (2 * TPACK_ROWS, D))

    g = ((xt >> 11) << 11) + ((xt & 1023) << 1) + ((xt >> 10) & 1)
    idx3 = g.reshape(-1)

    out3 = None
    j_off = 0
    for nj_split in NJ_SPLITS:
        rows = nj_split * NI
        r0 = j_off * NI
        idx_s = lax.slice(idx3, (r0,), (r0 + rows,))
        out2 = _make_gather(rows)(idx_s, tview)
        r3 = jnp.reshape(out2, (nj_split, NQ, 128))
        sel = _make_select(nj_split, j_off, aliased=out3 is not None)
        out3 = sel(r3) if out3 is None else sel(r3, out3)
        j_off += nj_split
    return jnp.transpose(out3, (2, 0, 1))


# speedup vs baseline: 1.8218x; 1.1767x over previous
import jax
import jax.numpy as jnp
from jax import lax
from jax.experimental import pallas as pl
from jax.experimental.pallas import tpu as pltpu
from jax.experimental.pallas import tpu_sc as plsc

VOCAB = 1000000
D = 64
NI, NJ = 4096, 200
B = NI * NJ
NC, NS = 2, 16
NW = NC * NS

W1 = 1024
G1 = (VOCAB + 2 * W1 - 1) // (2 * W1)
TPACK_ROWS2 = (G1 // 2 + 1) * 2 * W1


def _repack_body(a_ref, b_ref, c_ref, d_ref, o_ref):
    o_ref[0:W1, 0:D] = jnp.transpose(a_ref[...])
    o_ref[0:W1, D:] = jnp.transpose(b_ref[...])
    o_ref[W1:, 0:D] = jnp.transpose(c_ref[...])
    o_ref[W1:, D:] = jnp.transpose(d_ref[...])


def _win(k):
    return lambda i: (0, jnp.minimum(4 * i + k, 976))


_repack = pl.pallas_call(
    _repack_body,
    grid=(G1 // 2 + 1,),
    in_specs=[pl.BlockSpec((D, W1), _win(k)) for k in range(4)],
    out_specs=pl.BlockSpec((2 * W1, 128), lambda i: (i, 0)),
    out_shape=jax.ShapeDtypeStruct((TPACK_ROWS2, 128), jnp.float32),
)

CHUNK = 1024


def _gather_body(idx_hbm, tview_hbm, out_hbm, idx_v, rows_v, sem):
    n_chunks_total = idx_hbm.shape[0] // CHUNK
    per_w = n_chunks_total // NW
    wid = lax.axis_index("s") * NC + lax.axis_index("c")

    def step(i, carry):
        t = wid * per_w + i
        j = t >> 2
        s = (t >> 1) & 1
        c = t & 1
        pltpu.sync_copy(
            idx_hbm.at[pl.ds(j * NI + s * NQ + c * CHUNK, CHUNK)], idx_v
        )
        pltpu.async_copy(tview_hbm.at[idx_v], rows_v, sem).wait()
        pltpu.sync_copy(
            rows_v,
            out_hbm.at[pl.ds(j * NQ + c * CHUNK, CHUNK), pl.ds(s * D, D)],
        )
        return carry

    lax.fori_loop(0, per_w, step, 0)


def _make_gather(nrows):
    return pl.kernel(
        _gather_body,
        out_type=jax.ShapeDtypeStruct((nrows // 2, 2 * D), jnp.float32),
        mesh=plsc.VectorSubcoreMesh(core_axis_name="c", subcore_axis_name="s"),
        compiler_params=pltpu.CompilerParams(use_tc_tiling_on_sc=False),
        scratch_types=[
            pltpu.VMEM((CHUNK,), jnp.int32),
            pltpu.VMEM((CHUNK, D), jnp.float32),
            pltpu.SemaphoreType.DMA,
        ],
    )


NQ = NI // 2
NJ_SPLITS = (48, 48, 48, 56)


def _select_t_body(r_ref, o_ref):
    blk = r_ref[...]
    tblk = jnp.transpose(blk, (0, 2, 1))
    o_ref[...] = jnp.concatenate([tblk[:, :D, :], tblk[:, D:, :]], axis=2)


def _select_t_body_aliased(r_ref, _prev_ref, o_ref):
    _select_t_body(r_ref, o_ref)


_OUT3_TYPE = jax.ShapeDtypeStruct((NJ, D, NI), jnp.float32)


def _make_select(nj_split, j_off, aliased):
    in_specs = [pl.BlockSpec((1, NQ, 128), lambda a: (a, 0, 0))]
    kwargs = {}
    body = _select_t_body
    if aliased:
        in_specs.append(pl.BlockSpec(memory_space=pl.ANY))
        kwargs["input_output_aliases"] = {1: 0}
        body = _select_t_body_aliased
    return pl.pallas_call(
        body,
        grid=(nj_split,),
        in_specs=in_specs,
        out_specs=pl.BlockSpec((1, D, NI), lambda a: (a + j_off, 0, 0)),
        out_shape=_OUT3_TYPE,
        **kwargs,
    )


def kernel(x, table):
    tT = jnp.transpose(table)
    xt = jnp.transpose(x).astype(jnp.int32)
    tpack = _repack(tT, tT, tT, tT)
    tview = jnp.reshape(tpack, (2 * TPACK_ROWS2, D))

    g = ((xt >> 11) << 11) + ((xt & 1023) << 1) + ((xt >> 10) & 1)
    idx3 = g.reshape(-1)

    out3 = None
    j_off = 0
    for nj_split in NJ_SPLITS:
        rows = nj_split * NI
        r0 = j_off * NI
        idx_s = lax.slice(idx3, (r0,), (r0 + rows,))
        out2 = _make_gather(rows)(idx_s, tview)
        r3 = jnp.reshape(out2, (nj_split, NQ, 128))
        sel = _make_select(nj_split, j_off, aliased=out3 is not None)
        out3 = sel(r3) if out3 is None else sel(r3, out3)
        j_off += nj_split
    return jnp.transpose(out3, (2, 0, 1))


# speedup vs baseline: 1.8626x; 1.0224x over previous
import jax
import jax.numpy as jnp
from jax import lax
from jax.experimental import pallas as pl
from jax.experimental.pallas import tpu as pltpu
from jax.experimental.pallas import tpu_sc as plsc

VOCAB = 1000000
D = 64
NI, NJ = 4096, 200
B = NI * NJ
NC, NS = 2, 16
NW = NC * NS

W1 = 1024
G1 = (VOCAB + 2 * W1 - 1) // (2 * W1)
TPACK_ROWS2 = (G1 // 2 + 1) * 2 * W1


def _repack_body(a_ref, b_ref, c_ref, d_ref, o_ref):
    o_ref[0:W1, 0:D] = jnp.transpose(a_ref[...])
    o_ref[0:W1, D:] = jnp.transpose(b_ref[...])
    o_ref[W1:, 0:D] = jnp.transpose(c_ref[...])
    o_ref[W1:, D:] = jnp.transpose(d_ref[...])


def _win(k):
    return lambda i: (0, jnp.minimum(4 * i + k, 976))


_repack = pl.pallas_call(
    _repack_body,
    grid=(G1 // 2 + 1,),
    in_specs=[pl.BlockSpec((D, W1), _win(k)) for k in range(4)],
    out_specs=pl.BlockSpec((2 * W1, 128), lambda i: (i, 0)),
    out_shape=jax.ShapeDtypeStruct((TPACK_ROWS2, 128), jnp.float32),
)

CHUNK = 1024


def _gather_body(idx_hbm, tview_hbm, out_hbm, idx_v, rows_v, sem):
    n_chunks_total = idx_hbm.shape[0] // CHUNK
    per_w = n_chunks_total // NW
    wid = lax.axis_index("s") * NC + lax.axis_index("c")

    def step(i, carry):
        t = wid * per_w + i
        j = t >> 2
        s = (t >> 1) & 1
        c = t & 1
        pltpu.sync_copy(
            idx_hbm.at[pl.ds(j * NI + s * NQ + c * CHUNK, CHUNK)], idx_v
        )
        pltpu.async_copy(tview_hbm.at[idx_v], rows_v, sem).wait()
        pltpu.sync_copy(
            rows_v,
            out_hbm.at[pl.ds(j * NQ + c * CHUNK, CHUNK), pl.ds(s * D, D)],
        )
        return carry

    lax.fori_loop(0, per_w, step, 0)


def _make_gather(nrows):
    return pl.kernel(
        _gather_body,
        out_type=jax.ShapeDtypeStruct((nrows // 2, 2 * D), jnp.float32),
        mesh=plsc.VectorSubcoreMesh(core_axis_name="c", subcore_axis_name="s"),
        compiler_params=pltpu.CompilerParams(use_tc_tiling_on_sc=False),
        scratch_types=[
            pltpu.VMEM((CHUNK,), jnp.int32),
            pltpu.VMEM((CHUNK, D), jnp.float32),
            pltpu.SemaphoreType.DMA,
        ],
    )


NQ = NI // 2
NJ_SPLITS = (48, 48, 48, 56)


JB = 2


def _select_t_body(r_ref, o_ref):
    for jj in range(JB):
        o_ref[jj, :, :NQ] = jnp.transpose(r_ref[jj, :, :D])
        o_ref[jj, :, NQ:] = jnp.transpose(r_ref[jj, :, D:])


def _select_t_body_aliased(r_ref, _prev_ref, o_ref):
    _select_t_body(r_ref, o_ref)


_OUT3_TYPE = jax.ShapeDtypeStruct((NJ, D, NI), jnp.float32)


def _make_select(nj_split, j_off, aliased):
    in_specs = [pl.BlockSpec((JB, NQ, 128), lambda a: (a, 0, 0))]
    kwargs = {}
    body = _select_t_body
    if aliased:
        in_specs.append(pl.BlockSpec(memory_space=pl.ANY))
        kwargs["input_output_aliases"] = {1: 0}
        body = _select_t_body_aliased
    return pl.pallas_call(
        body,
        grid=(nj_split // JB,),
        in_specs=in_specs,
        out_specs=pl.BlockSpec(
            (JB, D, NI), lambda a: (a + j_off // JB, 0, 0)
        ),
        out_shape=_OUT3_TYPE,
        **kwargs,
    )


def kernel(x, table):
    tT = jnp.transpose(table)
    xt = jnp.transpose(x).astype(jnp.int32)
    tpack = _repack(tT, tT, tT, tT)
    tview = jnp.reshape(tpack, (2 * TPACK_ROWS2, D))

    g = ((xt >> 11) << 11) + ((xt & 1023) << 1) + ((xt >> 10) & 1)
    idx3 = g.reshape(-1)

    out3 = None
    j_off = 0
    for nj_split in NJ_SPLITS:
        rows = nj_split * NI
        r0 = j_off * NI
        idx_s = lax.slice(idx3, (r0,), (r0 + rows,))
        out2 = _make_gather(rows)(idx_s, tview)
        r3 = jnp.reshape(out2, (nj_split, NQ, 128))
        sel = _make_select(nj_split, j_off, aliased=out3 is not None)
        out3 = sel(r3) if out3 is None else sel(r3, out3)
        j_off += nj_split
    return jnp.transpose(out3, (2, 0, 1))


# speedup vs baseline: 2.0613x; 1.1067x over previous
import jax
import jax.numpy as jnp
from jax import lax
from jax.experimental import pallas as pl
from jax.experimental.pallas import tpu as pltpu
from jax.experimental.pallas import tpu_sc as plsc

VOCAB = 1000000
D = 64
NI, NJ = 4096, 200
B = NI * NJ
NC, NS = 2, 16
NW = NC * NS

W1 = 1024
G1 = (VOCAB + 2 * W1 - 1) // (2 * W1)
NWIN = 8
TPACK_ROWS2 = ((2 * G1 + NWIN - 1) // NWIN) * NWIN * W1 // 2


def _repack_body(*refs):
    o_ref = refs[-1]
    for k in range(NWIN):
        r0 = (k // 2) * W1
        c0 = (k % 2) * D
        o_ref[r0:r0 + W1, c0:c0 + D] = jnp.transpose(refs[k][...])


def _win(k):
    return lambda i: (0, jnp.minimum(NWIN * i + k, 976))


_repack = pl.pallas_call(
    _repack_body,
    grid=((2 * G1 + NWIN - 1) // NWIN,),
    in_specs=[pl.BlockSpec((D, W1), _win(k)) for k in range(NWIN)],
    out_specs=pl.BlockSpec((NWIN * W1 // 2, 128), lambda i: (i, 0)),
    out_shape=jax.ShapeDtypeStruct((TPACK_ROWS2, 128), jnp.float32),
)

CHUNK = 1024


def _gather_body(idx_hbm, tview_hbm, out_hbm, idx_v, rows_v, sem):
    n_chunks_total = idx_hbm.shape[0] // CHUNK
    per_w = n_chunks_total // NW
    wid = lax.axis_index("s") * NC + lax.axis_index("c")

    def step(i, carry):
        t = wid * per_w + i
        j = t >> 2
        s = (t >> 1) & 1
        c = t & 1
        pltpu.sync_copy(
            idx_hbm.at[pl.ds(j * NI + s * NQ + c * CHUNK, CHUNK)], idx_v
        )
        pltpu.async_copy(tview_hbm.at[idx_v], rows_v, sem).wait()
        pltpu.sync_copy(
            rows_v,
            out_hbm.at[pl.ds(j * NQ + c * CHUNK, CHUNK), pl.ds(s * D, D)],
        )
        return carry

    lax.fori_loop(0, per_w, step, 0)


def _make_gather(nrows):
    return pl.kernel(
        _gather_body,
        out_type=jax.ShapeDtypeStruct((nrows // 2, 2 * D), jnp.float32),
        mesh=plsc.VectorSubcoreMesh(core_axis_name="c", subcore_axis_name="s"),
        compiler_params=pltpu.CompilerParams(use_tc_tiling_on_sc=False),
        scratch_types=[
            pltpu.VMEM((CHUNK,), jnp.int32),
            pltpu.VMEM((CHUNK, D), jnp.float32),
            pltpu.SemaphoreType.DMA,
        ],
    )


NQ = NI // 2
NJ_SPLITS = (48, 48, 48, 56)


JB = 2


def _select_t_body(r_ref, o_ref):
    for jj in range(JB):
        o_ref[jj, :, :NQ] = jnp.transpose(r_ref[jj, :, :D])
        o_ref[jj, :, NQ:] = jnp.transpose(r_ref[jj, :, D:])


def _select_t_body_aliased(r_ref, _prev_ref, o_ref):
    _select_t_body(r_ref, o_ref)


_OUT3_TYPE = jax.ShapeDtypeStruct((NJ, D, NI), jnp.float32)


def _make_select(nj_split, j_off, aliased):
    in_specs = [pl.BlockSpec((JB, NQ, 128), lambda a: (a, 0, 0))]
    kwargs = {}
    body = _select_t_body
    if aliased:
        in_specs.append(pl.BlockSpec(memory_space=pl.ANY))
        kwargs["input_output_aliases"] = {1: 0}
        body = _select_t_body_aliased
    return pl.pallas_call(
        body,
        grid=(nj_split // JB,),
        in_specs=in_specs,
        out_specs=pl.BlockSpec(
            (JB, D, NI), lambda a: (a + j_off // JB, 0, 0)
        ),
        out_shape=_OUT3_TYPE,
        **kwargs,
    )


def kernel(x, table):
    tT = jnp.transpose(table)
    xt = jnp.transpose(x).astype(jnp.int32)
    tpack = _repack(*([tT] * NWIN))
    tview = jnp.reshape(tpack, (2 * TPACK_ROWS2, D))

    g = ((xt >> 11) << 11) + ((xt & 1023) << 1) + ((xt >> 10) & 1)
    idx3 = g.reshape(-1)

    out3 = None
    j_off = 0
    for nj_split in NJ_SPLITS:
        rows = nj_split * NI
        r0 = j_off * NI
        idx_s = lax.slice(idx3, (r0,), (r0 + rows,))
        out2 = _make_gather(rows)(idx_s, tview)
        r3 = jnp.reshape(out2, (nj_split, NQ, 128))
        sel = _make_select(nj_split, j_off, aliased=out3 is not None)
        out3 = sel(r3) if out3 is None else sel(r3, out3)
        j_off += nj_split
    return jnp.transpose(out3, (2, 0, 1))


# speedup vs baseline: 2.1362x; 1.0364x over previous
import jax
import jax.numpy as jnp
from jax import lax
from jax.experimental import pallas as pl
from jax.experimental.pallas import tpu as pltpu
from jax.experimental.pallas import tpu_sc as plsc

VOCAB = 1000000
D = 64
NI, NJ = 4096, 200
B = NI * NJ
NC, NS = 2, 16
NW = NC * NS

W1 = 1024
G1 = (VOCAB + 2 * W1 - 1) // (2 * W1)
NWIN = 16
TPACK_ROWS2 = ((2 * G1 + NWIN - 1) // NWIN) * NWIN * W1 // 2


def _repack_body(*refs):
    o_ref = refs[-1]
    for k in range(NWIN):
        r0 = (k // 2) * W1
        c0 = (k % 2) * D
        o_ref[r0:r0 + W1, c0:c0 + D] = jnp.transpose(refs[k][...])


def _win(k):
    return lambda i: (0, jnp.minimum(NWIN * i + k, 976))


_repack = pl.pallas_call(
    _repack_body,
    grid=((2 * G1 + NWIN - 1) // NWIN,),
    in_specs=[pl.BlockSpec((D, W1), _win(k)) for k in range(NWIN)],
    out_specs=pl.BlockSpec((NWIN * W1 // 2, 128), lambda i: (i, 0)),
    out_shape=jax.ShapeDtypeStruct((TPACK_ROWS2, 128), jnp.float32),
)

CHUNK = 1024


def _gather_body(idx_hbm, tview_hbm, out_hbm, idx_v, rows_v, sem):
    n_chunks_total = idx_hbm.shape[0] // CHUNK
    per_w = n_chunks_total // NW
    wid = lax.axis_index("s") * NC + lax.axis_index("c")

    def step(i, carry):
        t = wid * per_w + i
        j = t >> 2
        s = (t >> 1) & 1
        c = t & 1
        pltpu.sync_copy(
            idx_hbm.at[pl.ds(j * NI + s * NQ + c * CHUNK, CHUNK)], idx_v
        )
        pltpu.async_copy(tview_hbm.at[idx_v], rows_v, sem).wait()
        pltpu.sync_copy(
            rows_v,
            out_hbm.at[pl.ds(j * NQ + c * CHUNK, CHUNK), pl.ds(s * D, D)],
        )
        return carry

    lax.fori_loop(0, per_w, step, 0)


def _make_gather(nrows):
    return pl.kernel(
        _gather_body,
        out_type=jax.ShapeDtypeStruct((nrows // 2, 2 * D), jnp.float32),
        mesh=plsc.VectorSubcoreMesh(core_axis_name="c", subcore_axis_name="s"),
        compiler_params=pltpu.CompilerParams(use_tc_tiling_on_sc=False),
        scratch_types=[
            pltpu.VMEM((CHUNK,), jnp.int32),
            pltpu.VMEM((CHUNK, D), jnp.float32),
            pltpu.SemaphoreType.DMA,
        ],
    )


NQ = NI // 2
NJ_SPLITS = (48, 48, 48, 56)


JB = 2


def _select_t_body(r_ref, o_ref):
    for jj in range(JB):
        o_ref[jj, :, :NQ] = jnp.transpose(r_ref[jj, :, :D])
        o_ref[jj, :, NQ:] = jnp.transpose(r_ref[jj, :, D:])


def _select_t_body_aliased(r_ref, _prev_ref, o_ref):
    _select_t_body(r_ref, o_ref)


_OUT3_TYPE = jax.ShapeDtypeStruct((NJ, D, NI), jnp.float32)


def _make_select(nj_split, j_off, aliased):
    in_specs = [pl.BlockSpec((JB, NQ, 128), lambda a: (a, 0, 0))]
    kwargs = {}
    body = _select_t_body
    if aliased:
        in_specs.append(pl.BlockSpec(memory_space=pl.ANY))
        kwargs["input_output_aliases"] = {1: 0}
        body = _select_t_body_aliased
    return pl.pallas_call(
        body,
        grid=(nj_split // JB,),
        in_specs=in_specs,
        out_specs=pl.BlockSpec(
            (JB, D, NI), lambda a: (a + j_off // JB, 0, 0)
        ),
        out_shape=_OUT3_TYPE,
        **kwargs,
    )


def kernel(x, table):
    tT = jnp.transpose(table)
    xt = jnp.transpose(x).astype(jnp.int32)
    tpack = _repack(*([tT] * NWIN))
    tview = jnp.reshape(tpack, (2 * TPACK_ROWS2, D))

    g = ((xt >> 11) << 11) + ((xt & 1023) << 1) + ((xt >> 10) & 1)
    idx3 = g.reshape(-1)

    out3 = None
    j_off = 0
    for nj_split in NJ_SPLITS:
        rows = nj_split * NI
        r0 = j_off * NI
        idx_s = lax.slice(idx3, (r0,), (r0 + rows,))
        out2 = _make_gather(rows)(idx_s, tview)
        r3 = jnp.reshape(out2, (nj_split, NQ, 128))
        sel = _make_select(nj_split, j_off, aliased=out3 is not None)
        out3 = sel(r3) if out3 is None else sel(r3, out3)
        j_off += nj_split
    return jnp.transpose(out3, (2, 0, 1))


# speedup vs baseline: 2.1548x; 1.0087x over previous
import jax
import jax.numpy as jnp
from jax import lax
from jax.experimental import pallas as pl
from jax.experimental.pallas import tpu as pltpu
from jax.experimental.pallas import tpu_sc as plsc

VOCAB = 1000000
D = 64
NI, NJ = 4096, 200
B = NI * NJ
NC, NS = 2, 16
NW = NC * NS

W1 = 1024
G1 = (VOCAB + 2 * W1 - 1) // (2 * W1)
NWIN = 32
TPACK_ROWS2 = ((2 * G1 + NWIN - 1) // NWIN) * NWIN * W1 // 2


def _repack_body(*refs):
    o_ref = refs[-1]
    for k in range(NWIN):
        r0 = (k // 2) * W1
        c0 = (k % 2) * D
        o_ref[r0:r0 + W1, c0:c0 + D] = jnp.transpose(refs[k][...])


def _win(k):
    return lambda i: (0, jnp.minimum(NWIN * i + k, 976))


_repack = pl.pallas_call(
    _repack_body,
    grid=((2 * G1 + NWIN - 1) // NWIN,),
    in_specs=[pl.BlockSpec((D, W1), _win(k)) for k in range(NWIN)],
    out_specs=pl.BlockSpec((NWIN * W1 // 2, 128), lambda i: (i, 0)),
    out_shape=jax.ShapeDtypeStruct((TPACK_ROWS2, 128), jnp.float32),
)

CHUNK = 1024


def _gather_body(idx_hbm, tview_hbm, out_hbm, idx_v, rows_v, sem):
    n_chunks_total = idx_hbm.shape[0] // CHUNK
    per_w = n_chunks_total // NW
    wid = lax.axis_index("s") * NC + lax.axis_index("c")

    def step(i, carry):
        t = wid * per_w + i
        j = t >> 2
        s = (t >> 1) & 1
        c = t & 1
        pltpu.sync_copy(
            idx_hbm.at[pl.ds(j * NI + s * NQ + c * CHUNK, CHUNK)], idx_v
        )
        pltpu.async_copy(tview_hbm.at[idx_v], rows_v, sem).wait()
        pltpu.sync_copy(
            rows_v,
            out_hbm.at[pl.ds(j * NQ + c * CHUNK, CHUNK), pl.ds(s * D, D)],
        )
        return carry

    lax.fori_loop(0, per_w, step, 0)


def _make_gather(nrows):
    return pl.kernel(
        _gather_body,
        out_type=jax.ShapeDtypeStruct((nrows // 2, 2 * D), jnp.float32),
        mesh=plsc.VectorSubcoreMesh(core_axis_name="c", subcore_axis_name="s"),
        compiler_params=pltpu.CompilerParams(use_tc_tiling_on_sc=False),
        scratch_types=[
            pltpu.VMEM((CHUNK,), jnp.int32),
            pltpu.VMEM((CHUNK, D), jnp.float32),
            pltpu.SemaphoreType.DMA,
        ],
    )


NQ = NI // 2
NJ_SPLITS = (48, 48, 48, 56)


JB = 4


def _select_t_body(r_ref, o_ref):
    for jj in range(JB):
        o_ref[jj, :, :NQ] = jnp.transpose(r_ref[jj, :, :D])
        o_ref[jj, :, NQ:] = jnp.transpose(r_ref[jj, :, D:])


def _select_t_body_aliased(r_ref, _prev_ref, o_ref):
    _select_t_body(r_ref, o_ref)


_OUT3_TYPE = jax.ShapeDtypeStruct((NJ, D, NI), jnp.float32)


def _make_select(nj_split, j_off, aliased):
    in_specs = [pl.BlockSpec((JB, NQ, 128), lambda a: (a, 0, 0))]
    kwargs = {}
    body = _select_t_body
    if aliased:
        in_specs.append(pl.BlockSpec(memory_space=pl.ANY))
        kwargs["input_output_aliases"] = {1: 0}
        body = _select_t_body_aliased
    return pl.pallas_call(
        body,
        grid=(nj_split // JB,),
        in_specs=in_specs,
        out_specs=pl.BlockSpec(
            (JB, D, NI), lambda a: (a + j_off // JB, 0, 0)
        ),
        out_shape=_OUT3_TYPE,
        **kwargs,
    )


def kernel(x, table):
    tT = jnp.transpose(table)
    xt = jnp.transpose(x).astype(jnp.int32)
    tpack = _repack(*([tT] * NWIN))
    tview = jnp.reshape(tpack, (2 * TPACK_ROWS2, D))

    g = ((xt >> 11) << 11) + ((xt & 1023) << 1) + ((xt >> 10) & 1)
    idx3 = g.reshape(-1)

    out3 = None
    j_off = 0
    for nj_split in NJ_SPLITS:
        rows = nj_split * NI
        r0 = j_off * NI
        idx_s = lax.slice(idx3, (r0,), (r0 + rows,))
        out2 = _make_gather(rows)(idx_s, tview)
        r3 = jnp.reshape(out2, (nj_split, NQ, 128))
        sel = _make_select(nj_split, j_off, aliased=out3 is not None)
        out3 = sel(r3) if out3 is None else sel(r3, out3)
        j_off += nj_split
    return jnp.transpose(out3, (2, 0, 1))
